# Initial kernel scaffold; baseline (speedup 1.0000x reference)
import jax
import jax.numpy as jnp
from jax.experimental import pallas as pl


def kernel(slots):
    raise NotImplementedError("write your pallas kernel here")


# speedup vs baseline: 4.0994x; 4.0994x over previous
import jax
import jax.numpy as jnp
from jax import lax
from jax.experimental import pallas as pl

_EPS = 1e-8
_THRESH = 0.9
_BB = 8


def _merge_block_kernel(slots_ref, final_ref, smask_ref):
    S = slots_ref.shape[1]
    j_iota = lax.broadcasted_iota(jnp.int32, (S, S), 1)
    s_iota = lax.broadcasted_iota(jnp.int32, (S, S), 0)
    eye_f = (s_iota == j_iota).astype(jnp.float32)
    for b in range(slots_ref.shape[0]):
        x = slots_ref[b]
        norm2 = jnp.sum(x * x, axis=1, keepdims=True)
        nn = jnp.sqrt(norm2)
        g = lax.dot_general(x, x, (((1,), (1,)), ((), ())),
                            preferred_element_type=jnp.float32)
        denom = nn * nn.reshape(1, S) + _EPS
        sim = g / denom
        maskb = sim > _THRESH
        maskf = maskb.astype(jnp.float32)
        count = jnp.sum(maskf, axis=1, keepdims=True)
        multi_f = (count > 1.0).astype(jnp.float32)
        first_idx = jnp.min(jnp.where(maskb, j_iota, S), axis=1,
                            keepdims=True)
        notfirst_f = (j_iota != first_idx).astype(jnp.float32)
        zc = maskf * notfirst_f * multi_f
        smask = 1.0 - jnp.max(zc, axis=0, keepdims=True)
        wm = multi_f * maskf + (1.0 - multi_f) * eye_f
        s_last = jnp.max(jnp.where(wm > 0.5, s_iota, -1), axis=0,
                         keepdims=True)
        oh = (s_last.reshape(S, 1) == j_iota).astype(jnp.float32)
        mhat = multi_f * (maskf / (count + _EPS)) + (1.0 - multi_f) * eye_f
        w = lax.dot_general(oh, mhat, (((1,), (0,)), ((), ())),
                            preferred_element_type=jnp.float32)
        final_ref[b] = lax.dot_general(w, x, (((1,), (0,)), ((), ())),
                                       preferred_element_type=jnp.float32)
        smask_ref[pl.ds(b, 1), :] = smask


def kernel(slots):
    B, S, D = slots.shape
    grid = (B // _BB,)
    final, smask = pl.pallas_call(
        _merge_block_kernel,
        grid=grid,
        in_specs=[pl.BlockSpec((_BB, S, D), lambda i: (i, 0, 0))],
        out_specs=[
            pl.BlockSpec((_BB, S, D), lambda i: (i, 0, 0)),
            pl.BlockSpec((_BB, S), lambda i: (i, 0)),
        ],
        out_shape=[
            jax.ShapeDtypeStruct((B, S, D), slots.dtype),
            jax.ShapeDtypeStruct((B, S), slots.dtype),
        ],
    )(slots)
    return final, smask


# speedup vs baseline: 6.6218x; 1.6153x over previous
import jax
import jax.numpy as jnp
from jax import lax
from jax.experimental import pallas as pl

_EPS = 1e-8
_THRESH = 0.9
_BB = 8


def _merge_block_kernel(slots_ref, final_ref, smask_ref):
    BB, S, D = slots_ref.shape
    N = BB * S
    X = slots_ref[...].reshape(N, D)
    lane = lax.broadcasted_iota(jnp.int32, (N, S), 1)
    rowid = lax.broadcasted_iota(jnp.int32, (N, 1), 0) & (S - 1)
    inv = lax.rsqrt(jnp.sum(X * X, axis=1, keepdims=True))
    Y = X * inv
    gs = []
    for b in range(BB):
        yb = Y[b * S:(b + 1) * S]
        gs.append(lax.dot_general(yb, yb, (((1,), (1,)), ((), ())),
                                  preferred_element_type=jnp.float32))
    G = jnp.concatenate(gs, axis=0)
    maskb = G > _THRESH
    maskf = maskb.astype(jnp.float32)
    count = jnp.sum(maskf, axis=1, keepdims=True)
    multi_f = (count > 1.0).astype(jnp.float32)
    fi = jnp.min(jnp.where(maskb, lane, S), axis=1, keepdims=True)
    cols = jnp.concatenate([multi_f, fi.astype(jnp.float32)], axis=1)
    mrows, frows = [], []
    for b in range(BB):
        t = jnp.transpose(cols[b * S:(b + 1) * S])
        mrows.append(jnp.broadcast_to(t[0:1, :], (S, S)))
        frows.append(jnp.broadcast_to(t[1:2, :], (S, S)))
    MR = jnp.concatenate(mrows, axis=0)
    FR = jnp.concatenate(frows, axis=0)
    rowid_f = rowid.astype(jnp.float32)
    zc = maskf * MR * (rowid_f != FR).astype(jnp.float32)
    smask_col = 1.0 - jnp.max(zc, axis=1, keepdims=True)
    wm = maskf * MR
    slw = jnp.max(jnp.where(wm > 0.5, lane, -1), axis=1, keepdims=True)
    s_last = jnp.where(multi_f > 0.5, slw, jnp.maximum(slw, rowid))
    oh = (s_last == lane).astype(jnp.float32)
    eye_st = (rowid == lane).astype(jnp.float32)
    mhat = multi_f * (maskf / (count + _EPS)) + (1.0 - multi_f) * eye_st
    for b in range(BB):
        sl = slice(b * S, (b + 1) * S)
        w = lax.dot_general(oh[sl], mhat[sl], (((1,), (0,)), ((), ())),
                            preferred_element_type=jnp.float32)
        final_ref[b] = lax.dot_general(w, X[sl], (((1,), (0,)), ((), ())),
                                       preferred_element_type=jnp.float32)
    smask_ref[...] = smask_col.reshape(BB, S)


def kernel(slots):
    B, S, D = slots.shape
    grid = (B // _BB,)
    final, smask = pl.pallas_call(
        _merge_block_kernel,
        grid=grid,
        in_specs=[pl.BlockSpec((_BB, S, D), lambda i: (i, 0, 0))],
        out_specs=[
            pl.BlockSpec((_BB, S, D), lambda i: (i, 0, 0)),
            pl.BlockSpec((_BB, S), lambda i: (i, 0)),
        ],
        out_shape=[
            jax.ShapeDtypeStruct((B, S, D), slots.dtype),
            jax.ShapeDtypeStruct((B, S), slots.dtype),
        ],
    )(slots)
    return final, smask


# speedup vs baseline: 8.2519x; 1.2462x over previous
import jax
import jax.numpy as jnp
from jax import lax
from jax.experimental import pallas as pl

_EPS = 1e-8
_THRESH = 0.9
_BB = 16


def _merge_block_kernel(slots_ref, final_ref, smask_ref):
    BB, S, D = slots_ref.shape
    N = BB * S
    X = slots_ref[...].reshape(N, D)
    lane = lax.broadcasted_iota(jnp.int32, (N, S), 1)
    rowid = lax.broadcasted_iota(jnp.int32, (N, 1), 0) & (S - 1)
    inv = lax.rsqrt(jnp.sum(X * X, axis=1, keepdims=True))
    Y = X * inv
    gs = []
    for b in range(BB):
        yb = Y[b * S:(b + 1) * S]
        gs.append(lax.dot_general(yb, yb, (((1,), (1,)), ((), ())),
                                  preferred_element_type=jnp.float32))
    G = jnp.concatenate(gs, axis=0)
    maskb = G > _THRESH
    maskf = maskb.astype(jnp.float32)
    count = jnp.sum(maskf, axis=1, keepdims=True)
    multi_f = (count > 1.0).astype(jnp.float32)
    fi = jnp.min(jnp.where(maskb, lane, S), axis=1, keepdims=True)
    cols = jnp.concatenate([multi_f, fi.astype(jnp.float32)], axis=1)
    mrows, frows = [], []
    for b in range(BB):
        t = jnp.transpose(cols[b * S:(b + 1) * S])
        mrows.append(jnp.broadcast_to(t[0:1, :], (S, S)))
        frows.append(jnp.broadcast_to(t[1:2, :], (S, S)))
    MR = jnp.concatenate(mrows, axis=0)
    FR = jnp.concatenate(frows, axis=0)
    rowid_f = rowid.astype(jnp.float32)
    zc = maskf * MR * (rowid_f != FR).astype(jnp.float32)
    smask_col = 1.0 - jnp.max(zc, axis=1, keepdims=True)
    wm = maskf * MR
    slw = jnp.max(jnp.where(wm > 0.5, lane, -1), axis=1, keepdims=True)
    s_last = jnp.where(multi_f > 0.5, slw, jnp.maximum(slw, rowid))
    oh = (s_last == lane).astype(jnp.float32)
    eye_st = (rowid == lane).astype(jnp.float32)
    mhat = multi_f * (maskf / (count + _EPS)) + (1.0 - multi_f) * eye_st
    for b in range(BB):
        sl = slice(b * S, (b + 1) * S)
        w = lax.dot_general(oh[sl], mhat[sl], (((1,), (0,)), ((), ())),
                            preferred_element_type=jnp.float32)
        final_ref[b] = lax.dot_general(w, X[sl], (((1,), (0,)), ((), ())),
                                       preferred_element_type=jnp.float32)
    smask_ref[...] = smask_col.reshape(BB, S)


def kernel(slots):
    B, S, D = slots.shape
    grid = (B // _BB,)
    final, smask = pl.pallas_call(
        _merge_block_kernel,
        grid=grid,
        in_specs=[pl.BlockSpec((_BB, S, D), lambda i: (i, 0, 0))],
        out_specs=[
            pl.BlockSpec((_BB, S, D), lambda i: (i, 0, 0)),
            pl.BlockSpec((_BB, S), lambda i: (i, 0)),
        ],
        out_shape=[
            jax.ShapeDtypeStruct((B, S, D), slots.dtype),
            jax.ShapeDtypeStruct((B, S), slots.dtype),
        ],
    )(slots)
    return final, smask


# speedup vs baseline: 8.8281x; 1.0698x over previous
import jax
import jax.numpy as jnp
from jax import lax
from jax.experimental import pallas as pl

_EPS = 1e-8
_THRESH = 0.9
_BB = 32


def _merge_block_kernel(slots_ref, final_ref, smask_ref):
    BB, S, D = slots_ref.shape
    N = BB * S
    X = slots_ref[...].reshape(N, D)
    lane = lax.broadcasted_iota(jnp.int32, (N, S), 1)
    rowid = lax.broadcasted_iota(jnp.int32, (N, 1), 0) & (S - 1)
    inv = lax.rsqrt(jnp.sum(X * X, axis=1, keepdims=True))
    Y = X * inv
    gs = []
    for b in range(BB):
        yb = Y[b * S:(b + 1) * S]
        gs.append(lax.dot_general(yb, yb, (((1,), (1,)), ((), ())),
                                  preferred_element_type=jnp.float32))
    G = jnp.concatenate(gs, axis=0)
    maskb = G > _THRESH
    maskf = maskb.astype(jnp.float32)
    count = jnp.sum(maskf, axis=1, keepdims=True)
    multi_f = (count > 1.0).astype(jnp.float32)
    fi = jnp.min(jnp.where(maskb, lane, S), axis=1, keepdims=True)
    cols = jnp.concatenate([multi_f, fi.astype(jnp.float32)], axis=1)
    mrows, frows = [], []
    for b in range(BB):
        t = jnp.transpose(cols[b * S:(b + 1) * S])
        mrows.append(jnp.broadcast_to(t[0:1, :], (S, S)))
        frows.append(jnp.broadcast_to(t[1:2, :], (S, S)))
    MR = jnp.concatenate(mrows, axis=0)
    FR = jnp.concatenate(frows, axis=0)
    rowid_f = rowid.astype(jnp.float32)
    zc = maskf * MR * (rowid_f != FR).astype(jnp.float32)
    smask_col = 1.0 - jnp.max(zc, axis=1, keepdims=True)
    wm = maskf * MR
    slw = jnp.max(jnp.where(wm > 0.5, lane, -1), axis=1, keepdims=True)
    s_last = jnp.where(multi_f > 0.5, slw, jnp.maximum(slw, rowid))
    oh = (s_last == lane).astype(jnp.float32)
    eye_st = (rowid == lane).astype(jnp.float32)
    mhat = multi_f * (maskf / (count + _EPS)) + (1.0 - multi_f) * eye_st
    for b in range(BB):
        sl = slice(b * S, (b + 1) * S)
        w = lax.dot_general(oh[sl], mhat[sl], (((1,), (0,)), ((), ())),
                            preferred_element_type=jnp.float32)
        final_ref[b] = lax.dot_general(w, X[sl], (((1,), (0,)), ((), ())),
                                       preferred_element_type=jnp.float32)
    smask_ref[...] = smask_col.reshape(BB, S)


def kernel(slots):
    B, S, D = slots.shape
    grid = (B // _BB,)
    final, smask = pl.pallas_call(
        _merge_block_kernel,
        grid=grid,
        in_specs=[pl.BlockSpec((_BB, S, D), lambda i: (i, 0, 0))],
        out_specs=[
            pl.BlockSpec((_BB, S, D), lambda i: (i, 0, 0)),
            pl.BlockSpec((_BB, S), lambda i: (i, 0)),
        ],
        out_shape=[
            jax.ShapeDtypeStruct((B, S, D), slots.dtype),
            jax.ShapeDtypeStruct((B, S), slots.dtype),
        ],
    )(slots)
    return final, smask
